# baseline (speedup 1.0000x reference)
import functools
import jax
import jax.numpy as jnp
from jax import lax
from jax.experimental import pallas as pl
from jax.experimental.pallas import tpu as pltpu
from jax.experimental.pallas import tpu_sc as plsc

B = 128
T = 1024
D = 128
VOCAB = 16
N_SEG = 8
MASK_ID = 5
N = B * T


R = 8192
RB = R // 128
NBLK = N // R


def _tc_block(tok_ref, msk_ref, seg_ref, tok_tab_ref, seg_tab_ref,
              tok_seg_ref, maskpos_ref):
    tok = tok_ref[...]
    seg = seg_ref[...]
    msk = msk_ref[...]

    tokp = (tok * N_SEG + seg).astype(jnp.float32)

    row_id = jax.lax.broadcasted_iota(jnp.int32, (R, RB), 0)
    grp_id = jax.lax.broadcasted_iota(jnp.int32, (R, RB), 1)
    E = (row_id // 128 == grp_id).astype(jnp.float32)
    rr = jax.lax.broadcasted_iota(jnp.int32, (R, 128), 0)
    cc = jax.lax.broadcasted_iota(jnp.int32, (R, 128), 1)
    Dm = (rr % 128 == cc).astype(jnp.float32)
    ones = jnp.ones((128, 1), jnp.float32)

    t1 = jnp.dot(E, tokp, preferred_element_type=jnp.float32)
    ftok = jnp.dot(t1 * Dm, ones, preferred_element_type=jnp.float32)

    ts = jax.lax.broadcasted_iota(jnp.int32, (VOCAB * N_SEG, VOCAB), 0)
    tv = jax.lax.broadcasted_iota(jnp.int32, (VOCAB * N_SEG, VOCAB), 1)
    E16 = ((ts // N_SEG == tv) & (ts // N_SEG != 0)).astype(jnp.float32)
    ss = jax.lax.broadcasted_iota(jnp.int32, (VOCAB * N_SEG, N_SEG), 0)
    sv = jax.lax.broadcasted_iota(jnp.int32, (VOCAB * N_SEG, N_SEG), 1)
    E8 = ((ss % N_SEG == sv) & (ss % N_SEG != 0)).astype(jnp.float32)
    combo_tok = jnp.dot(E16, tok_tab_ref[...], preferred_element_type=jnp.float32)
    combo_seg = jnp.dot(E8, seg_tab_ref[...], preferred_element_type=jnp.float32)

    iota128 = jax.lax.broadcasted_iota(jnp.int32, (R, VOCAB * N_SEG), 1)
    oh_tok = (ftok.astype(jnp.int32) == iota128).astype(jnp.float32)

    tok_seg_ref[:, :D] = jnp.dot(oh_tok, combo_tok, preferred_element_type=jnp.float32)
    tok_seg_ref[:, D:] = jnp.dot(oh_tok, combo_seg, preferred_element_type=jnp.float32)
    maskpos_ref[...] = (msk == MASK_ID).astype(jnp.int32)


def _tc_call(tok2, msk2, seg2, token_table, seg_table):
    out_shapes = (
        jax.ShapeDtypeStruct((N, 2 * D), jnp.float32),
        jax.ShapeDtypeStruct((N // 128, 128), jnp.int32),
    )
    ids_spec = pl.BlockSpec((RB, 128), lambda i: (i, 0))
    tab16_spec = pl.BlockSpec((VOCAB, D), lambda i: (0, 0))
    tab8_spec = pl.BlockSpec((N_SEG, D), lambda i: (0, 0))
    out_spec = pl.BlockSpec((R, 2 * D), lambda i: (i, 0))
    maskpos_spec = pl.BlockSpec((RB, 128), lambda i: (i, 0))
    return pl.pallas_call(
        _tc_block,
        grid=(NBLK,),
        in_specs=[ids_spec, ids_spec, ids_spec, tab16_spec, tab8_spec],
        out_specs=[out_spec, maskpos_spec],
        out_shape=out_shapes,
    )(tok2, msk2, seg2, token_table, seg_table)


NC = 2
NS = 16
NW = NC * NS
PER_W = N // NW
CHUNK = 64
NCHUNK = PER_W // CHUNK
NBUF = 4


def _sc_body(msk_hbm, seg_hbm, ctab_msk, out_msk,
             msk_v, seg_v, idx2, buf1, buf2, buf3, buf4, gsem, wsem):
    wid = lax.axis_index("s") * NC + lax.axis_index("c")
    w_base = wid * PER_W

    pltpu.sync_copy(msk_hbm.at[pl.ds(w_base, PER_W)], msk_v)
    pltpu.sync_copy(seg_hbm.at[pl.ds(w_base, PER_W)], seg_v)

    def idx_body(r, _):
        for k in range(CHUNK // 16):
            sl = pl.ds(r * CHUNK + k * 16, 16)
            idx2[r, pl.ds(k * 16, 16)] = msk_v[sl] * N_SEG + seg_v[sl]
        return ()

    lax.fori_loop(0, NCHUNK, idx_body, ())

    bufs = (buf1, buf2, buf3, buf4)
    pltpu.async_copy(ctab_msk.at[idx2.at[0]], bufs[0], gsem)
    pltpu.async_copy(ctab_msk.at[idx2.at[1]], bufs[1], gsem)

    def chunk_body(io, _):
        for b in range(NBUF):
            i = io * NBUF + b
            buf = bufs[b]
            nxt = bufs[(b + 2) % NBUF]
            pltpu.make_async_copy(ctab_msk.at[idx2.at[0]], buf, gsem).wait()

            @pl.when(i >= 2)
            def _():
                pltpu.make_async_copy(nxt, out_msk.at[pl.ds(w_base, CHUNK)], wsem).wait()

            @pl.when(i + 2 < NCHUNK)
            def _():
                pltpu.async_copy(ctab_msk.at[idx2.at[i + 2]], nxt, gsem)

            pltpu.async_copy(buf, out_msk.at[pl.ds(w_base + i * CHUNK, CHUNK)], wsem)
        return ()

    lax.fori_loop(0, NCHUNK // NBUF, chunk_body, ())
    pltpu.make_async_copy(bufs[2], out_msk.at[pl.ds(w_base, CHUNK)], wsem).wait()
    pltpu.make_async_copy(bufs[3], out_msk.at[pl.ds(w_base, CHUNK)], wsem).wait()


_mesh = plsc.VectorSubcoreMesh(core_axis_name="c", subcore_axis_name="s")

_sc_call = functools.partial(
    pl.kernel,
    mesh=_mesh,
    out_type=[
        jax.ShapeDtypeStruct((N, 2 * D), jnp.float32),
    ],
    scratch_types=[
        pltpu.VMEM((PER_W,), jnp.int32),
        pltpu.VMEM((PER_W,), jnp.int32),
        pltpu.VMEM((NCHUNK, CHUNK), jnp.int32),
        pltpu.VMEM((CHUNK, 2 * D), jnp.float32),
        pltpu.VMEM((CHUNK, 2 * D), jnp.float32),
        pltpu.VMEM((CHUNK, 2 * D), jnp.float32),
        pltpu.VMEM((CHUNK, 2 * D), jnp.float32),
        pltpu.SemaphoreType.DMA,
        pltpu.SemaphoreType.DMA,
    ],
)(_sc_body)


def kernel(token_table, mask_table, seg_table, region_tokens, region_tokens_mask, segment_ids, region_structures):
    tmask = (jnp.arange(VOCAB) != 0).astype(jnp.float32)[:, None]
    smask = (jnp.arange(N_SEG) != 0).astype(jnp.float32)[:, None]
    msk_rep = jnp.repeat(mask_table * tmask, N_SEG, axis=0)
    seg_rep = jnp.tile(seg_table * smask, (VOCAB, 1))
    ctab_msk = jnp.concatenate([msk_rep, seg_rep], axis=1)

    mskf = region_tokens_mask.reshape(N)
    segf = segment_ids.reshape(N)
    (msk_seg,) = _sc_call(mskf, segf, ctab_msk)

    tok2 = region_tokens.reshape(N // 128, 128)
    msk2 = region_tokens_mask.reshape(N // 128, 128)
    seg2 = segment_ids.reshape(N // 128, 128)
    tok_seg, maskpos = _tc_call(tok2, msk2, seg2, token_table, seg_table)

    tok_seg = tok_seg.reshape(B, T, 2 * D)
    msk_seg = msk_seg.reshape(B, T, 2 * D)
    mask_positions = maskpos.reshape(B, T).astype(jnp.bool_)
    return (tok_seg, msk_seg, region_tokens, region_structures, region_tokens_mask, mask_positions)


# speedup vs baseline: 3.3963x; 3.3963x over previous
import jax
import jax.numpy as jnp
from jax.experimental import pallas as pl

B = 128
T = 1024
D = 128
VOCAB = 16
N_SEG = 8
MASK_ID = 5
N = B * T
R = 8192
RB = R // 128
NBLK = N // R


def _emb_block(tok_ref, msk_ref, seg_ref, tok_tab_ref, msk_tab_ref, seg_tab_ref,
               tok_seg_ref, msk_seg_ref, maskpos_ref):
    tok = tok_ref[...]
    msk = msk_ref[...]
    seg = seg_ref[...]

    tokp = (tok * N_SEG + seg).astype(jnp.float32)
    mskp = (msk * N_SEG + seg).astype(jnp.float32)

    row_id = jax.lax.broadcasted_iota(jnp.int32, (R, RB), 0)
    grp_id = jax.lax.broadcasted_iota(jnp.int32, (R, RB), 1)
    E = (row_id // 128 == grp_id).astype(jnp.float32)
    rr = jax.lax.broadcasted_iota(jnp.int32, (R, 128), 0)
    cc = jax.lax.broadcasted_iota(jnp.int32, (R, 128), 1)
    Dm = (rr % 128 == cc).astype(jnp.float32)
    ones = jnp.ones((128, 1), jnp.float32)

    def flatten(x):
        t1 = jnp.dot(E, x, preferred_element_type=jnp.float32)
        return jnp.dot(t1 * Dm, ones, preferred_element_type=jnp.float32)

    ftok = flatten(tokp)
    fmsk = flatten(mskp)

    ts = jax.lax.broadcasted_iota(jnp.int32, (VOCAB * N_SEG, VOCAB), 0)
    tv = jax.lax.broadcasted_iota(jnp.int32, (VOCAB * N_SEG, VOCAB), 1)
    E16 = ((ts // N_SEG == tv) & (ts // N_SEG != 0)).astype(jnp.float32)
    ss = jax.lax.broadcasted_iota(jnp.int32, (VOCAB * N_SEG, N_SEG), 0)
    sv = jax.lax.broadcasted_iota(jnp.int32, (VOCAB * N_SEG, N_SEG), 1)
    E8 = ((ss % N_SEG == sv) & (ss % N_SEG != 0)).astype(jnp.float32)
    combo_tok = jnp.dot(E16, tok_tab_ref[...], preferred_element_type=jnp.float32)
    combo_msk = jnp.dot(E16, msk_tab_ref[...], preferred_element_type=jnp.float32)
    combo_seg = jnp.dot(E8, seg_tab_ref[...], preferred_element_type=jnp.float32)

    iota128 = jax.lax.broadcasted_iota(jnp.int32, (R, VOCAB * N_SEG), 1)
    oh_tok = (ftok.astype(jnp.int32) == iota128).astype(jnp.float32)
    oh_msk = (fmsk.astype(jnp.int32) == iota128).astype(jnp.float32)

    tok_seg_ref[:, :D] = jnp.dot(oh_tok, combo_tok, preferred_element_type=jnp.float32)
    tok_seg_ref[:, D:] = jnp.dot(oh_tok, combo_seg, preferred_element_type=jnp.float32)
    msk_seg_ref[:, :D] = jnp.dot(oh_msk, combo_msk, preferred_element_type=jnp.float32)
    msk_seg_ref[:, D:] = jnp.dot(oh_msk, combo_seg, preferred_element_type=jnp.float32)
    maskpos_ref[...] = (msk == MASK_ID).astype(jnp.int32)


def kernel(token_table, mask_table, seg_table, region_tokens, region_tokens_mask, segment_ids, region_structures):
    tok = region_tokens.reshape(N // 128, 128)
    msk = region_tokens_mask.reshape(N // 128, 128)
    seg = segment_ids.reshape(N // 128, 128)

    out_shapes = (
        jax.ShapeDtypeStruct((N, 2 * D), jnp.float32),
        jax.ShapeDtypeStruct((N, 2 * D), jnp.float32),
        jax.ShapeDtypeStruct((N // 128, 128), jnp.int32),
    )
    ids_spec = pl.BlockSpec((RB, 128), lambda i: (i, 0))
    tab16_spec = pl.BlockSpec((VOCAB, D), lambda i: (0, 0))
    tab8_spec = pl.BlockSpec((N_SEG, D), lambda i: (0, 0))
    out_spec = pl.BlockSpec((R, 2 * D), lambda i: (i, 0))
    maskpos_spec = pl.BlockSpec((RB, 128), lambda i: (i, 0))

    tok_seg, msk_seg, maskpos = pl.pallas_call(
        _emb_block,
        grid=(NBLK,),
        in_specs=[ids_spec, ids_spec, ids_spec, tab16_spec, tab16_spec, tab8_spec],
        out_specs=[out_spec, out_spec, maskpos_spec],
        out_shape=out_shapes,
    )(tok, msk, seg, token_table, mask_table, seg_table)

    tok_seg = tok_seg.reshape(B, T, 2 * D)
    msk_seg = msk_seg.reshape(B, T, 2 * D)
    mask_positions = maskpos.reshape(B, T).astype(jnp.bool_)
    return (tok_seg, msk_seg, region_tokens, region_structures, region_tokens_mask, mask_positions)
